# baseline (speedup 1.0000x reference)
import functools

import jax
import jax.numpy as jnp
from jax import lax
from jax.experimental import pallas as pl
from jax.experimental.pallas import tpu as pltpu
from jax.experimental.pallas import tpu_sc as plsc

_N = 10000
_E = 320000
_D = 128
_OUT = 40

_NW = 32
_K = 64
_NCA = 189
_NCB = 135
_ROWS = 16 * (_NCA + _NCB)
_EPAD = _ROWS * _K
_NPAD = 10240
_SLICE = _NPAD // 16
_BR = 1024
_GRID = _NPAD // _BR


def _alphas(h, asr, adr):
    return jnp.concatenate(
        [jnp.sum(h * asr, axis=1)[None, :], jnp.sum(h * adr, axis=1)[None, :]],
        axis=0)


def _head1_body(x_ref, w_ref, asr_ref, adr_ref, h_ref, al_ref):
    sl = pl.ds(pl.program_id(0) * _BR, _BR)
    h = jnp.dot(x_ref[...], w_ref[...], preferred_element_type=jnp.float32)
    h_ref[...] = h
    al_ref[:, sl] = _alphas(h, asr_ref[...], adr_ref[...])


def _normed_in(aA_ref, aB_ref, den_ref, b_ref):
    sl = pl.ds(pl.program_id(0) * _BR, _BR)
    rden = 1.0 / (den_ref[0, sl] + den_ref[1, sl])
    xx = (aA_ref[...] + aB_ref[...]) * rden[:, None] + b_ref[...]
    return jnp.maximum(xx, 0.0)


def _head2_body(aA_ref, aB_ref, den_ref, b_ref, w_ref, asr_ref, adr_ref,
                h_ref, al_ref):
    sl = pl.ds(pl.program_id(0) * _BR, _BR)
    xx = _normed_in(aA_ref, aB_ref, den_ref, b_ref)
    h = jnp.dot(xx, w_ref[...], preferred_element_type=jnp.float32)
    h_ref[...] = h
    al_ref[:, sl] = _alphas(h, asr_ref[...], adr_ref[...])


def _head3_body(aA_ref, aB_ref, den_ref, b_ref, w_ref, bl_ref, out_ref):
    xx = _normed_in(aA_ref, aB_ref, den_ref, b_ref)
    out_ref[...] = (jnp.dot(xx, w_ref[...], preferred_element_type=jnp.float32)
                    + bl_ref[...])


def _head1(x, W, asr, adr):
    return pl.pallas_call(
        _head1_body,
        grid=(_GRID,),
        in_specs=[
            pl.BlockSpec((_BR, _D), lambda i: (i, 0)),
            pl.BlockSpec((_D, _D), lambda i: (0, 0)),
            pl.BlockSpec((1, _D), lambda i: (0, 0)),
            pl.BlockSpec((1, _D), lambda i: (0, 0)),
        ],
        out_specs=[
            pl.BlockSpec((_BR, _D), lambda i: (i, 0)),
            pl.BlockSpec((2, _NPAD), lambda i: (0, 0)),
        ],
        out_shape=[
            jax.ShapeDtypeStruct((_NPAD, _D), jnp.float32),
            jax.ShapeDtypeStruct((2, _NPAD), jnp.float32),
        ],
    )(x, W, asr, adr)


def _head2(aA, aB, den, b, W, asr, adr):
    return pl.pallas_call(
        _head2_body,
        grid=(_GRID,),
        in_specs=[
            pl.BlockSpec((_BR, _D), lambda i: (i, 0)),
            pl.BlockSpec((_BR, _D), lambda i: (i + _GRID, 0)),
            pl.BlockSpec((2, _NPAD), lambda i: (0, 0)),
            pl.BlockSpec((1, _D), lambda i: (0, 0)),
            pl.BlockSpec((_D, _D), lambda i: (0, 0)),
            pl.BlockSpec((1, _D), lambda i: (0, 0)),
            pl.BlockSpec((1, _D), lambda i: (0, 0)),
        ],
        out_specs=[
            pl.BlockSpec((_BR, _D), lambda i: (i, 0)),
            pl.BlockSpec((2, _NPAD), lambda i: (0, 0)),
        ],
        out_shape=[
            jax.ShapeDtypeStruct((_NPAD, _D), jnp.float32),
            jax.ShapeDtypeStruct((2, _NPAD), jnp.float32),
        ],
    )(aA, aB, den, b, W, asr, adr)


def _head3(aA, aB, den, b, W, bl):
    return pl.pallas_call(
        _head3_body,
        grid=(_GRID,),
        in_specs=[
            pl.BlockSpec((_BR, _D), lambda i: (i, 0)),
            pl.BlockSpec((_BR, _D), lambda i: (i + _GRID, 0)),
            pl.BlockSpec((2, _NPAD), lambda i: (0, 0)),
            pl.BlockSpec((1, _D), lambda i: (0, 0)),
            pl.BlockSpec((_D, _OUT), lambda i: (0, 0)),
            pl.BlockSpec((1, _OUT), lambda i: (0, 0)),
        ],
        out_specs=pl.BlockSpec((_BR, _OUT), lambda i: (i, 0)),
        out_shape=jax.ShapeDtypeStruct((_NPAD, _OUT), jnp.float32),
    )(aA, aB, den, b, W, bl)


def _edge_body(pk_hbm, h_hbm, as_hbm, ad_hbm,
               acc_hbm, den_hbm,
               pkr_v, srow_v, drow_v, av_v, bv_v, ee_v, rows_v, zden_v,
               acc_s, den_s,
               sg0, sg1, sg2, ss0, ss1, ss2, sp0, sp1, sp2,
               sl0, sl1, sl2):
    cid = lax.axis_index("c")
    sid = lax.axis_index("s")
    sem_g = (sg0, sg1, sg2)
    sem_s = (ss0, ss1, ss2)
    sem_p = (sp0, sp1, sp2)
    sem_gl = (sl0, sl1, sl2)

    zero16 = jnp.zeros((16,), jnp.float32)

    @pl.loop(0, _K)
    def _zr(i):
        for c in range(8):
            rows_v[0, i, pl.ds(c * 16, 16)] = zero16

    @pl.loop(0, 40)
    def _zd(i):
        zden_v[pl.ds(i * 16, 16)] = zero16

    for k in range(_SLICE // _K):
        pltpu.sync_copy(rows_v.at[0],
                        acc_s.at[pl.ds(sid * _SLICE + k * _K, _K)])
    pltpu.sync_copy(rows_v.at[0, pl.ds(0, _SLICE % _K)],
                    acc_s.at[pl.ds(sid * _SLICE + (_SLICE // _K) * _K,
                                   _SLICE % _K)])
    pltpu.sync_copy(zden_v, den_s.at[pl.ds(sid * _SLICE, _SLICE)])

    nc = jnp.where(cid == 0, _NCA, _NCB)
    rowbase = jnp.where(cid == 0, sid * _NCA, 16 * _NCA + sid * _NCB)

    def _pk_off(jj):
        return pl.multiple_of((rowbase + jj) * _K, 8)

    def _issue_pk(jj, b):
        pltpu.async_copy(pk_hbm.at[pl.ds(_pk_off(jj), _K)], pkr_v.at[b],
                         sem_p[b])

    def _wait_pk(jj, b):
        pltpu.make_async_copy(pk_hbm.at[pl.ds(_pk_off(jj), _K)], pkr_v.at[b],
                              sem_p[b]).wait()

    plsc.subcore_barrier()

    def _unpack(b):
        for t in range(_K // 16):
            sl = pl.ds(t * 16, 16)
            pk = pkr_v[b, sl]
            srow_v[b, sl] = pk >> 14
            drow_v[b, sl] = pk & 16383

    def _issue_gathers(b):
        pltpu.async_copy(h_hbm.at[srow_v.at[b]], rows_v.at[b], sem_g[b])
        pltpu.async_copy(as_hbm.at[srow_v.at[b]], av_v.at[b], sem_gl[b])
        pltpu.async_copy(ad_hbm.at[drow_v.at[b]], bv_v.at[b], sem_gl[b])

    def _wait_logits(b):
        pltpu.make_async_copy(as_hbm.at[srow_v.at[b]], av_v.at[b],
                              sem_gl[b]).wait()
        pltpu.make_async_copy(ad_hbm.at[drow_v.at[b]], bv_v.at[b],
                              sem_gl[b]).wait()

    def _wait_rows(b):
        pltpu.make_async_copy(h_hbm.at[srow_v.at[b]], rows_v.at[b],
                              sem_g[b]).wait()

    def _issue_scatters(b):
        pltpu.async_copy(ee_v.at[b], den_s.at[drow_v.at[b]], sem_s[b],
                         add=True)
        pltpu.async_copy(rows_v.at[b], acc_s.at[drow_v.at[b]], sem_s[b],
                         add=True)

    def _wait_scatters(b):
        pltpu.make_async_copy(ee_v.at[b], den_s.at[drow_v.at[b]],
                              sem_s[b]).wait()
        pltpu.make_async_copy(rows_v.at[b], acc_s.at[drow_v.at[b]],
                              sem_s[b]).wait()

    pltpu.sync_copy(pk_hbm.at[pl.ds(_pk_off(0), _K)], pkr_v.at[0])
    _unpack(0)
    _issue_gathers(0)
    _issue_pk(1, 1)

    @pl.loop(0, nc, step=3)
    def _pc(j0):
        for b in range(3):
            j = j0 + b
            nb = (b + 1) % 3

            pb = (b + 2) % 3

            @pl.when(j + 2 < nc)
            def _pkpre():
                _issue_pk(j + 2, pb)

            @pl.when(j + 1 < nc)
            def _prefetch():
                @pl.when(j >= 2)
                def _w():
                    _wait_scatters(nb)
                _wait_pk(j + 1, nb)
                _unpack(nb)
                _issue_gathers(nb)

            _wait_logits(b)
            for t in range(_K // 16):
                sl = pl.ds(t * 16, 16)
                e = av_v[b, sl] + bv_v[b, sl]
                e = jnp.where(e >= 0.0, e, 0.2 * e)
                ee_v[b, sl] = jnp.exp(e)
            _wait_rows(b)

            @plsc.parallel_loop(0, _K // 16)
            def _mul(g):
                eev = ee_v[b, pl.ds(g * 16, 16)]
                for r in range(16):
                    a = eev[r]
                    for c in range(8):
                        sl = pl.ds(c * 16, 16)
                        rows_v[b, g * 16 + r, sl] = (
                            rows_v[b, g * 16 + r, sl] * a)

            _issue_scatters(b)

    for b in range(3):
        _wait_scatters(b)

    plsc.subcore_barrier()

    base = cid * _NPAD + sid * _SLICE
    pltpu.sync_copy(acc_s.at[pl.ds(sid * _SLICE, _SLICE)],
                    acc_hbm.at[pl.ds(base, _SLICE)])
    @pl.when(sid == 0)
    def _wden():
        pltpu.sync_copy(den_s, den_hbm.at[pl.ds(cid * _NPAD, _NPAD)])


_edge = functools.partial(
    pl.kernel,
    out_type=[
        jax.ShapeDtypeStruct((2 * _NPAD, _D), jnp.float32),
        jax.ShapeDtypeStruct((2 * _NPAD,), jnp.float32),
    ],
    mesh=plsc.VectorSubcoreMesh(core_axis_name="c", subcore_axis_name="s"),
    compiler_params=pltpu.CompilerParams(needs_layout_passes=False),
    scratch_types=[
        pltpu.VMEM((3, _K), jnp.int32),
        pltpu.VMEM((3, _K), jnp.int32),
        pltpu.VMEM((3, _K), jnp.int32),
        pltpu.VMEM((3, _K), jnp.float32),
        pltpu.VMEM((3, _K), jnp.float32),
        pltpu.VMEM((3, _K), jnp.float32),
        pltpu.VMEM((3, _K, _D), jnp.float32),
        pltpu.VMEM((640,), jnp.float32),
        pltpu.VMEM_SHARED((_NPAD, _D), jnp.float32),
        pltpu.VMEM_SHARED((_NPAD,), jnp.float32),
        pltpu.SemaphoreType.DMA,
        pltpu.SemaphoreType.DMA,
        pltpu.SemaphoreType.DMA,
        pltpu.SemaphoreType.DMA,
        pltpu.SemaphoreType.DMA,
        pltpu.SemaphoreType.DMA,
        pltpu.SemaphoreType.DMA,
        pltpu.SemaphoreType.DMA,
        pltpu.SemaphoreType.DMA,
        pltpu.SemaphoreType.DMA,
        pltpu.SemaphoreType.DMA,
        pltpu.SemaphoreType.DMA,
    ],
)(_edge_body)


def kernel(x, edge_index, W1, a_src1, a_dst1, b1, W2, a_src2, a_dst2, b2,
           Wl, bl):
    loops = jnp.arange(_N, dtype=jnp.int32)
    pad = _EPAD - _E - _N
    src = jnp.concatenate([edge_index[0].astype(jnp.int32), loops,
                           jnp.zeros((pad,), jnp.int32)])
    dst = jnp.concatenate([edge_index[1].astype(jnp.int32), loops,
                           jnp.full((pad,), _N, jnp.int32)])
    pk3 = (src << 14) | dst

    xp = jnp.pad(x, ((0, _NPAD - _N), (0, 0)))
    h1, al1 = _head1(xp, W1, a_src1[None, :], a_dst1[None, :])
    acc1, den1 = _edge(pk3, h1, al1[0], al1[1])

    h2, al2 = _head2(acc1, acc1, den1.reshape(2, _NPAD),
                     b1[None, :], W2, a_src2[None, :], a_dst2[None, :])
    acc2, den2 = _edge(pk3, h2, al2[0], al2[1])

    pred = _head3(acc2, acc2, den2.reshape(2, _NPAD),
                  b2[None, :], Wl, bl[None, :])
    return pred[:_N]


# speedup vs baseline: 1.0296x; 1.0296x over previous
import functools

import jax
import jax.numpy as jnp
from jax import lax
from jax.experimental import pallas as pl
from jax.experimental.pallas import tpu as pltpu
from jax.experimental.pallas import tpu_sc as plsc

_N = 10000
_E = 320000
_D = 128
_OUT = 40

_NW = 32
_K = 64
_NCA = 201
_NCB = 123
_ROWS = 16 * (_NCA + _NCB)
_EPAD = _ROWS * _K
_NPAD = 10240
_SLICE = _NPAD // 16
_BR = 1024
_GRID = _NPAD // _BR


def _alphas(h, asr, adr):
    return jnp.concatenate(
        [jnp.sum(h * asr, axis=1)[None, :], jnp.sum(h * adr, axis=1)[None, :]],
        axis=0)


def _head1_body(x_ref, w_ref, asr_ref, adr_ref, h_ref, al_ref):
    sl = pl.ds(pl.program_id(0) * _BR, _BR)
    h = jnp.dot(x_ref[...], w_ref[...], preferred_element_type=jnp.float32)
    h_ref[...] = h
    al_ref[:, sl] = _alphas(h, asr_ref[...], adr_ref[...])


def _normed_in(aA_ref, aB_ref, den_ref, b_ref):
    sl = pl.ds(pl.program_id(0) * _BR, _BR)
    rden = 1.0 / (den_ref[0, sl] + den_ref[1, sl])
    xx = (aA_ref[...] + aB_ref[...]) * rden[:, None] + b_ref[...]
    return jnp.maximum(xx, 0.0)


def _head2_body(aA_ref, aB_ref, den_ref, b_ref, w_ref, asr_ref, adr_ref,
                h_ref, al_ref):
    sl = pl.ds(pl.program_id(0) * _BR, _BR)
    xx = _normed_in(aA_ref, aB_ref, den_ref, b_ref)
    h = jnp.dot(xx, w_ref[...], preferred_element_type=jnp.float32)
    h_ref[...] = h
    al_ref[:, sl] = _alphas(h, asr_ref[...], adr_ref[...])


def _head3_body(aA_ref, aB_ref, den_ref, b_ref, w_ref, bl_ref, out_ref):
    xx = _normed_in(aA_ref, aB_ref, den_ref, b_ref)
    out_ref[...] = (jnp.dot(xx, w_ref[...], preferred_element_type=jnp.float32)
                    + bl_ref[...])


def _head1(x, W, asr, adr):
    return pl.pallas_call(
        _head1_body,
        grid=(_GRID,),
        in_specs=[
            pl.BlockSpec((_BR, _D), lambda i: (i, 0)),
            pl.BlockSpec((_D, _D), lambda i: (0, 0)),
            pl.BlockSpec((1, _D), lambda i: (0, 0)),
            pl.BlockSpec((1, _D), lambda i: (0, 0)),
        ],
        out_specs=[
            pl.BlockSpec((_BR, _D), lambda i: (i, 0)),
            pl.BlockSpec((2, _NPAD), lambda i: (0, 0)),
        ],
        out_shape=[
            jax.ShapeDtypeStruct((_NPAD, _D), jnp.float32),
            jax.ShapeDtypeStruct((2, _NPAD), jnp.float32),
        ],
    )(x, W, asr, adr)


def _head2(aA, aB, den, b, W, asr, adr):
    return pl.pallas_call(
        _head2_body,
        grid=(_GRID,),
        in_specs=[
            pl.BlockSpec((_BR, _D), lambda i: (i, 0)),
            pl.BlockSpec((_BR, _D), lambda i: (i + _GRID, 0)),
            pl.BlockSpec((2, _NPAD), lambda i: (0, 0)),
            pl.BlockSpec((1, _D), lambda i: (0, 0)),
            pl.BlockSpec((_D, _D), lambda i: (0, 0)),
            pl.BlockSpec((1, _D), lambda i: (0, 0)),
            pl.BlockSpec((1, _D), lambda i: (0, 0)),
        ],
        out_specs=[
            pl.BlockSpec((_BR, _D), lambda i: (i, 0)),
            pl.BlockSpec((2, _NPAD), lambda i: (0, 0)),
        ],
        out_shape=[
            jax.ShapeDtypeStruct((_NPAD, _D), jnp.float32),
            jax.ShapeDtypeStruct((2, _NPAD), jnp.float32),
        ],
    )(aA, aB, den, b, W, asr, adr)


def _head3(aA, aB, den, b, W, bl):
    return pl.pallas_call(
        _head3_body,
        grid=(_GRID,),
        in_specs=[
            pl.BlockSpec((_BR, _D), lambda i: (i, 0)),
            pl.BlockSpec((_BR, _D), lambda i: (i + _GRID, 0)),
            pl.BlockSpec((2, _NPAD), lambda i: (0, 0)),
            pl.BlockSpec((1, _D), lambda i: (0, 0)),
            pl.BlockSpec((_D, _OUT), lambda i: (0, 0)),
            pl.BlockSpec((1, _OUT), lambda i: (0, 0)),
        ],
        out_specs=pl.BlockSpec((_BR, _OUT), lambda i: (i, 0)),
        out_shape=jax.ShapeDtypeStruct((_NPAD, _OUT), jnp.float32),
    )(aA, aB, den, b, W, bl)


def _edge_body(pk_hbm, h_hbm, as_hbm, ad_hbm,
               acc_hbm, den_hbm,
               pkr_v, srow_v, drow_v, av_v, bv_v, ee_v, rows_v, zden_v,
               acc_s, den_s,
               sg0, sg1, sg2, ss0, ss1, ss2, sp0, sp1, sp2,
               sl0, sl1, sl2):
    cid = lax.axis_index("c")
    sid = lax.axis_index("s")
    sem_g = (sg0, sg1, sg2)
    sem_s = (ss0, ss1, ss2)
    sem_p = (sp0, sp1, sp2)
    sem_gl = (sl0, sl1, sl2)

    zero16 = jnp.zeros((16,), jnp.float32)

    @pl.loop(0, _K)
    def _zr(i):
        for c in range(8):
            rows_v[0, i, pl.ds(c * 16, 16)] = zero16

    @pl.loop(0, 40)
    def _zd(i):
        zden_v[pl.ds(i * 16, 16)] = zero16

    for k in range(_SLICE // _K):
        pltpu.sync_copy(rows_v.at[0],
                        acc_s.at[pl.ds(sid * _SLICE + k * _K, _K)])
    pltpu.sync_copy(rows_v.at[0, pl.ds(0, _SLICE % _K)],
                    acc_s.at[pl.ds(sid * _SLICE + (_SLICE // _K) * _K,
                                   _SLICE % _K)])
    pltpu.sync_copy(zden_v, den_s.at[pl.ds(sid * _SLICE, _SLICE)])

    nc = jnp.where(cid == 0, _NCA, _NCB)
    rowbase = jnp.where(cid == 0, sid * _NCA, 16 * _NCA + sid * _NCB)

    def _pk_off(jj):
        return pl.multiple_of((rowbase + jj) * _K, 8)

    def _issue_pk(jj, b):
        pltpu.async_copy(pk_hbm.at[pl.ds(_pk_off(jj), _K)], pkr_v.at[b],
                         sem_p[b])

    def _wait_pk(jj, b):
        pltpu.make_async_copy(pk_hbm.at[pl.ds(_pk_off(jj), _K)], pkr_v.at[b],
                              sem_p[b]).wait()

    plsc.subcore_barrier()

    def _unpack(b):
        for t in range(_K // 16):
            sl = pl.ds(t * 16, 16)
            pk = pkr_v[b, sl]
            srow_v[b, sl] = pk >> 14
            drow_v[b, sl] = pk & 16383

    def _issue_gathers(b):
        pltpu.async_copy(h_hbm.at[srow_v.at[b]], rows_v.at[b], sem_g[b])
        pltpu.async_copy(as_hbm.at[srow_v.at[b]], av_v.at[b], sem_gl[b])
        pltpu.async_copy(ad_hbm.at[drow_v.at[b]], bv_v.at[b], sem_gl[b])

    def _wait_logits(b):
        pltpu.make_async_copy(as_hbm.at[srow_v.at[b]], av_v.at[b],
                              sem_gl[b]).wait()
        pltpu.make_async_copy(ad_hbm.at[drow_v.at[b]], bv_v.at[b],
                              sem_gl[b]).wait()

    def _wait_rows(b):
        pltpu.make_async_copy(h_hbm.at[srow_v.at[b]], rows_v.at[b],
                              sem_g[b]).wait()

    def _issue_scatters(b):
        pltpu.async_copy(ee_v.at[b], den_s.at[drow_v.at[b]], sem_s[b],
                         add=True)
        pltpu.async_copy(rows_v.at[b], acc_s.at[drow_v.at[b]], sem_s[b],
                         add=True)

    def _wait_scatters(b):
        pltpu.make_async_copy(ee_v.at[b], den_s.at[drow_v.at[b]],
                              sem_s[b]).wait()
        pltpu.make_async_copy(rows_v.at[b], acc_s.at[drow_v.at[b]],
                              sem_s[b]).wait()

    pltpu.sync_copy(pk_hbm.at[pl.ds(_pk_off(0), _K)], pkr_v.at[0])
    _unpack(0)
    _issue_gathers(0)
    _issue_pk(1, 1)

    @pl.loop(0, nc, step=3)
    def _pc(j0):
        for b in range(3):
            j = j0 + b
            nb = (b + 1) % 3

            pb = (b + 2) % 3

            @pl.when(j + 2 < nc)
            def _pkpre():
                _issue_pk(j + 2, pb)

            @pl.when(j + 1 < nc)
            def _prefetch():
                @pl.when(j >= 2)
                def _w():
                    _wait_scatters(nb)
                _wait_pk(j + 1, nb)
                _unpack(nb)
                _issue_gathers(nb)

            _wait_logits(b)
            for t in range(_K // 16):
                sl = pl.ds(t * 16, 16)
                e = av_v[b, sl] + bv_v[b, sl]
                e = jnp.where(e >= 0.0, e, 0.2 * e)
                ee_v[b, sl] = jnp.exp(e)
            _wait_rows(b)

            @plsc.parallel_loop(0, _K // 16)
            def _mul(g):
                eev = ee_v[b, pl.ds(g * 16, 16)]
                for r in range(16):
                    a = eev[r]
                    for c in range(8):
                        sl = pl.ds(c * 16, 16)
                        rows_v[b, g * 16 + r, sl] = (
                            rows_v[b, g * 16 + r, sl] * a)

            _issue_scatters(b)

    for b in range(3):
        _wait_scatters(b)

    plsc.subcore_barrier()

    base = cid * _NPAD + sid * _SLICE
    pltpu.sync_copy(acc_s.at[pl.ds(sid * _SLICE, _SLICE)],
                    acc_hbm.at[pl.ds(base, _SLICE)])
    @pl.when(sid == 0)
    def _wden():
        pltpu.sync_copy(den_s, den_hbm.at[pl.ds(cid * _NPAD, _NPAD)])


_edge = functools.partial(
    pl.kernel,
    out_type=[
        jax.ShapeDtypeStruct((2 * _NPAD, _D), jnp.float32),
        jax.ShapeDtypeStruct((2 * _NPAD,), jnp.float32),
    ],
    mesh=plsc.VectorSubcoreMesh(core_axis_name="c", subcore_axis_name="s"),
    compiler_params=pltpu.CompilerParams(needs_layout_passes=False),
    scratch_types=[
        pltpu.VMEM((3, _K), jnp.int32),
        pltpu.VMEM((3, _K), jnp.int32),
        pltpu.VMEM((3, _K), jnp.int32),
        pltpu.VMEM((3, _K), jnp.float32),
        pltpu.VMEM((3, _K), jnp.float32),
        pltpu.VMEM((3, _K), jnp.float32),
        pltpu.VMEM((3, _K, _D), jnp.float32),
        pltpu.VMEM((640,), jnp.float32),
        pltpu.VMEM_SHARED((_NPAD, _D), jnp.float32),
        pltpu.VMEM_SHARED((_NPAD,), jnp.float32),
        pltpu.SemaphoreType.DMA,
        pltpu.SemaphoreType.DMA,
        pltpu.SemaphoreType.DMA,
        pltpu.SemaphoreType.DMA,
        pltpu.SemaphoreType.DMA,
        pltpu.SemaphoreType.DMA,
        pltpu.SemaphoreType.DMA,
        pltpu.SemaphoreType.DMA,
        pltpu.SemaphoreType.DMA,
        pltpu.SemaphoreType.DMA,
        pltpu.SemaphoreType.DMA,
        pltpu.SemaphoreType.DMA,
    ],
)(_edge_body)


def kernel(x, edge_index, W1, a_src1, a_dst1, b1, W2, a_src2, a_dst2, b2,
           Wl, bl):
    loops = jnp.arange(_N, dtype=jnp.int32)
    pad = _EPAD - _E - _N
    src = jnp.concatenate([edge_index[0].astype(jnp.int32), loops,
                           jnp.zeros((pad,), jnp.int32)])
    dst = jnp.concatenate([edge_index[1].astype(jnp.int32), loops,
                           jnp.full((pad,), _N, jnp.int32)])
    pk3 = (src << 14) | dst

    xp = jnp.pad(x, ((0, _NPAD - _N), (0, 0)))
    h1, al1 = _head1(xp, W1, a_src1[None, :], a_dst1[None, :])
    acc1, den1 = _edge(pk3, h1, al1[0], al1[1])

    h2, al2 = _head2(acc1, acc1, den1.reshape(2, _NPAD),
                     b1[None, :], W2, a_src2[None, :], a_dst2[None, :])
    acc2, den2 = _edge(pk3, h2, al2[0], al2[1])

    pred = _head3(acc2, acc2, den2.reshape(2, _NPAD),
                  b2[None, :], Wl, bl[None, :])
    return pred[:_N]
